# Initial kernel scaffold; baseline (speedup 1.0000x reference)
import jax
import jax.numpy as jnp
from jax.experimental import pallas as pl


def kernel(x, ei, b, W0, c0, W1, c1, W2, c2, g0, be0, g1, be1, g2, be2, Wh1, bh1, Wh2, bh2):
    raise NotImplementedError("write your pallas kernel here")


# speedup vs baseline: 13.0014x; 13.0014x over previous
import functools

import jax
import jax.numpy as jnp
from jax import lax
from jax.experimental import pallas as pl
from jax.experimental.pallas import tpu as pltpu
from jax.experimental.pallas import tpu_sc as plsc

N = 10000
E = 160000
DIN = 256
H = 256
DOUT = 64
G = 64
EPS = 1e-5

NC = 2
NS = 16
L = 16
HH = H // 2
CHUNK = 128

BN = 1000
NB = N // BN

_mesh = lambda: plsc.VectorSubcoreMesh(core_axis_name="c", subcore_axis_name="s")


def _fill(ref, n, val):
  v = jnp.full((L,), val, jnp.float32)

  def body(i, _):
    ref[pl.ds(i * L, L)] = v
    return 0

  lax.fori_loop(0, n // L, body, 0)


def _sc_deg_body(dst_hbm, out_hbm, hist, ones_v, idx_v, zbuf):
  c = lax.axis_index("c")
  s = lax.axis_index("s")
  w = c * NS + s
  epw = 4992
  nfull = epw // CHUNK

  _fill(zbuf, CHUNK, 0.0)
  _fill(ones_v, CHUNK, 1.0)

  stripe = s * 624
  for k in range(5):
    pltpu.sync_copy(zbuf, hist.at[pl.ds(stripe + k * CHUNK, CHUNK)])
  plsc.subcore_barrier()

  base0 = w * epw

  def chunk(base):
    pltpu.sync_copy(dst_hbm.at[pl.ds(base, CHUNK)], idx_v)
    pltpu.sync_copy(ones_v, hist.at[idx_v], add=True)

  def body(i, _):
    chunk(base0 + i * CHUNK)
    return 0

  lax.fori_loop(0, nfull, body, 0)

  @pl.when(s == 0)
  def _():
    chunk(NC * NS * epw + c * CHUNK)

  plsc.subcore_barrier()

  @pl.when(s == 0)
  def _():
    pltpu.sync_copy(hist, out_hbm.at[c])


def _sc_deg(dst):
  return pl.kernel(
      _sc_deg_body,
      out_type=jax.ShapeDtypeStruct((NC, N), jnp.float32),
      mesh=_mesh(),
      scratch_types=[
          pltpu.VMEM_SHARED((N,), jnp.float32),
          pltpu.VMEM((CHUNK,), jnp.float32),
          pltpu.VMEM((CHUNK,), jnp.int32),
          pltpu.VMEM((CHUNK,), jnp.float32),
      ],
  )(dst)


def _sc_agg_body(src_hbm, dst_hbm, zs_hbm, out_hbm, acc, idx_s, idx_d,
                 idx_st, idx_dt, rows, rows_t, sems, sem_t):
  c = lax.axis_index("c")
  s = lax.axis_index("s")
  ept = E // NS
  nfull = ept // CHUNK
  tail = ept - nfull * CHUNK
  base0 = s * ept

  zv = jnp.zeros((L,), jnp.float32)

  def zbody(r, _):
    for jj in range(HH // L):
      rows[0, r, pl.ds(jj * L, L)] = zv
    return 0

  lax.fori_loop(0, CHUNK, zbody, 0)
  r0 = s * 624
  for k in range(5):
    pltpu.sync_copy(rows.at[0], acc.at[pl.ds(r0 + k * CHUNK, CHUNK)])
  plsc.subcore_barrier()

  def stage(i, slot):
    base = base0 + i * CHUNK
    pltpu.sync_copy(src_hbm.at[pl.ds(base, CHUNK)], idx_s.at[slot])
    pltpu.sync_copy(dst_hbm.at[pl.ds(base, CHUNK)], idx_d.at[slot])
    pltpu.async_copy(zs_hbm.at[c].at[idx_s.at[slot]], rows.at[slot],
                     sems.at[slot])

  stage(0, 0)

  def body(i, _):
    slot = lax.rem(i, 2)
    nxt = lax.rem(i + 1, 2)

    @pl.when(i + 1 < nfull)
    def _():
      stage(i + 1, nxt)

    pltpu.make_async_copy(zs_hbm.at[c].at[idx_s.at[slot]], rows.at[slot],
                          sems.at[slot]).wait()
    pltpu.sync_copy(rows.at[slot], acc.at[idx_d.at[slot]], add=True)
    return 0

  lax.fori_loop(0, nfull, body, 0)

  baset = base0 + nfull * CHUNK
  pltpu.sync_copy(src_hbm.at[pl.ds(baset, tail)], idx_st)
  pltpu.sync_copy(dst_hbm.at[pl.ds(baset, tail)], idx_dt)
  pltpu.async_copy(zs_hbm.at[c].at[idx_st], rows_t, sem_t).wait()
  pltpu.sync_copy(rows_t, acc.at[idx_dt], add=True)

  plsc.subcore_barrier()
  ro = s * 624

  @pl.when(s < NS - 1)
  def _():
    pltpu.sync_copy(acc.at[pl.ds(ro, 624)], out_hbm.at[c].at[pl.ds(ro, 624)])

  @pl.when(s == NS - 1)
  def _():
    pltpu.sync_copy(acc.at[pl.ds(9360, 640)],
                    out_hbm.at[c].at[pl.ds(9360, 640)])


def _sc_agg(src, dst, zs):
  return pl.kernel(
      _sc_agg_body,
      out_type=jax.ShapeDtypeStruct((NC, N, HH), jnp.float32),
      mesh=_mesh(),
      scratch_types=[
          pltpu.VMEM_SHARED((N, HH), jnp.float32),
          pltpu.VMEM((2, CHUNK), jnp.int32),
          pltpu.VMEM((2, CHUNK), jnp.int32),
          pltpu.VMEM((16,), jnp.int32),
          pltpu.VMEM((16,), jnp.int32),
          pltpu.VMEM((2, CHUNK, HH), jnp.float32),
          pltpu.VMEM((16, HH), jnp.float32),
          pltpu.SemaphoreType.DMA((2,)),
          pltpu.SemaphoreType.DMA,
      ],
  )(src, dst, zs)


def _tc_prep_body(x_ref, w_ref, da_ref, db_ref, zs_ref, dinv_ref):
  dinv = lax.rsqrt(da_ref[...] + db_ref[...] + 1.0)
  z = lax.dot_general(x_ref[...], w_ref[...], (((1,), (1,)), ((), ())),
                      preferred_element_type=jnp.float32)
  zs = z * dinv
  zs_ref[0] = zs[:, :HH]
  zs_ref[1] = zs[:, HH:]
  dinv_ref[...] = dinv


def _tc_prep(x, W0, degA, degB):
  return pl.pallas_call(
      _tc_prep_body,
      grid=(NB,),
      in_specs=[
          pl.BlockSpec((BN, DIN), lambda j: (j, 0)),
          pl.BlockSpec((H, DIN), lambda j: (0, 0)),
          pl.BlockSpec((BN, 1), lambda j: (j, 0)),
          pl.BlockSpec((BN, 1), lambda j: (j, 0)),
      ],
      out_specs=[
          pl.BlockSpec((NC, BN, HH), lambda j: (0, j, 0)),
          pl.BlockSpec((BN, 1), lambda j: (j, 0)),
      ],
      out_shape=[
          jax.ShapeDtypeStruct((NC, N, HH), jnp.float32),
          jax.ShapeDtypeStruct((N, 1), jnp.float32),
      ],
  )(x, W0, degA, degB)


def _pre_act(acc_ref, zs_ref, dinv_ref, c_ref):
  a = jnp.concatenate([acc_ref[0] + zs_ref[0], acc_ref[1] + zs_ref[1]],
                      axis=-1)
  return a * dinv_ref[...] + c_ref[...]


def _bn_relu(t, sums_ref, sumsq_ref, g_ref, be_ref):
  mean = sums_ref[...] * (1.0 / N)
  var = sumsq_ref[...] * (1.0 / N) - mean * mean
  return jnp.maximum((t - mean) * lax.rsqrt(var + EPS) * g_ref[...]
                     + be_ref[...], 0.0)


def _tc_layer_body(acc_ref, zs_ref, dinv_ref, c_ref, g_ref, be_ref, w_ref,
                   out_ref, sums_ref, sumsq_ref):
  p = pl.program_id(0)
  j = pl.program_id(1)
  t = _pre_act(acc_ref, zs_ref, dinv_ref, c_ref)

  @pl.when(p == 0)
  def _():
    @pl.when(j == 0)
    def _():
      sums_ref[...] = jnp.zeros_like(sums_ref)
      sumsq_ref[...] = jnp.zeros_like(sumsq_ref)

    sums_ref[...] += jnp.sum(t, axis=0, keepdims=True)
    sumsq_ref[...] += jnp.sum(t * t, axis=0, keepdims=True)

  @pl.when(p == 1)
  def _():
    h = _bn_relu(t, sums_ref, sumsq_ref, g_ref, be_ref)
    z = lax.dot_general(h, w_ref[...], (((1,), (1,)), ((), ())),
                        preferred_element_type=jnp.float32)
    zsn = z * dinv_ref[...]
    out_ref[0] = zsn[:, :HH]
    out_ref[1] = zsn[:, HH:]


def _tc_layer(acc, zs, dinv, c, g, be, Wn):
  return pl.pallas_call(
      _tc_layer_body,
      grid=(2, NB),
      in_specs=[
          pl.BlockSpec((NC, BN, HH), lambda p, j: (0, j, 0)),
          pl.BlockSpec((NC, BN, HH), lambda p, j: (0, j, 0)),
          pl.BlockSpec((BN, 1), lambda p, j: (j, 0)),
          pl.BlockSpec((1, H), lambda p, j: (0, 0)),
          pl.BlockSpec((1, H), lambda p, j: (0, 0)),
          pl.BlockSpec((1, H), lambda p, j: (0, 0)),
          pl.BlockSpec((H, H), lambda p, j: (0, 0)),
      ],
      out_specs=pl.BlockSpec((NC, BN, HH), lambda p, j: (0, j, 0)),
      out_shape=jax.ShapeDtypeStruct((NC, N, HH), jnp.float32),
      scratch_shapes=[
          pltpu.VMEM((1, H), jnp.float32),
          pltpu.VMEM((1, H), jnp.float32),
      ],
  )(acc, zs, dinv, c, g, be, Wn)


def _tc_head_body(acc_ref, zs_ref, dinv_ref, c_ref, g_ref, be_ref, b_ref,
                  wh1_ref, bh1_ref, wh2_ref, bh2_ref, out_ref,
                  sums_ref, sumsq_ref, psum_ref, pcnt_ref):
  p = pl.program_id(0)
  j = pl.program_id(1)

  @pl.when(p < 2)
  def _():
    t = _pre_act(acc_ref, zs_ref, dinv_ref, c_ref)

    @pl.when(p == 0)
    def _():
      @pl.when(j == 0)
      def _():
        sums_ref[...] = jnp.zeros_like(sums_ref)
        sumsq_ref[...] = jnp.zeros_like(sumsq_ref)

      sums_ref[...] += jnp.sum(t, axis=0, keepdims=True)
      sumsq_ref[...] += jnp.sum(t * t, axis=0, keepdims=True)

    @pl.when(p == 1)
    def _():
      @pl.when(j == 0)
      def _():
        psum_ref[...] = jnp.zeros_like(psum_ref)
        pcnt_ref[...] = jnp.zeros_like(pcnt_ref)

      h = _bn_relu(t, sums_ref, sumsq_ref, g_ref, be_ref)
      gids = lax.broadcasted_iota(jnp.int32, (1, G), 1)
      mask = (b_ref[...] == gids).astype(jnp.float32)
      psum_ref[...] += lax.dot_general(
          mask, h, (((0,), (0,)), ((), ())),
          preferred_element_type=jnp.float32)
      pcnt_ref[...] += lax.dot_general(
          mask, jnp.ones((BN, 1), jnp.float32), (((0,), (0,)), ((), ())),
          preferred_element_type=jnp.float32)

  @pl.when((p == 2) & (j == 0))
  def _():
    pool = psum_ref[...] / jnp.maximum(pcnt_ref[...], 1.0)
    hid = jnp.maximum(
        lax.dot_general(pool, wh1_ref[...], (((1,), (1,)), ((), ())),
                        preferred_element_type=jnp.float32) + bh1_ref[...],
        0.0)
    out_ref[...] = lax.dot_general(
        hid, wh2_ref[...], (((1,), (1,)), ((), ())),
        preferred_element_type=jnp.float32) + bh2_ref[...]


def _tc_head(acc, zs, dinv, c, g, be, b2, Wh1, bh1, Wh2, bh2):
  return pl.pallas_call(
      _tc_head_body,
      grid=(3, NB),
      in_specs=[
          pl.BlockSpec((NC, BN, HH), lambda p, j: (0, j, 0)),
          pl.BlockSpec((NC, BN, HH), lambda p, j: (0, j, 0)),
          pl.BlockSpec((BN, 1), lambda p, j: (j, 0)),
          pl.BlockSpec((1, H), lambda p, j: (0, 0)),
          pl.BlockSpec((1, H), lambda p, j: (0, 0)),
          pl.BlockSpec((1, H), lambda p, j: (0, 0)),
          pl.BlockSpec((BN, 1), lambda p, j: (j, 0)),
          pl.BlockSpec((H, H), lambda p, j: (0, 0)),
          pl.BlockSpec((1, H), lambda p, j: (0, 0)),
          pl.BlockSpec((DOUT, H), lambda p, j: (0, 0)),
          pl.BlockSpec((1, DOUT), lambda p, j: (0, 0)),
      ],
      out_specs=pl.BlockSpec((G, DOUT), lambda p, j: (0, 0)),
      out_shape=jax.ShapeDtypeStruct((G, DOUT), jnp.float32),
      scratch_shapes=[
          pltpu.VMEM((1, H), jnp.float32),
          pltpu.VMEM((1, H), jnp.float32),
          pltpu.VMEM((G, H), jnp.float32),
          pltpu.VMEM((G, 1), jnp.float32),
      ],
  )(acc, zs, dinv, c, g, be, b2, Wh1, bh1, Wh2, bh2)


def kernel(x, ei, b, W0, c0, W1, c1, W2, c2, g0, be0, g1, be1, g2, be2,
           Wh1, bh1, Wh2, bh2):
  src = ei[0]
  dst = ei[1]
  deg2 = _sc_deg(dst)
  degA = deg2[0].reshape(N, 1)
  degB = deg2[1].reshape(N, 1)
  zs, dinv = _tc_prep(x, W0, degA, degB)
  acc = _sc_agg(src, dst, zs)
  zs = _tc_layer(acc, zs, dinv, c0.reshape(1, H), g0.reshape(1, H),
                 be0.reshape(1, H), W1)
  acc = _sc_agg(src, dst, zs)
  zs = _tc_layer(acc, zs, dinv, c1.reshape(1, H), g1.reshape(1, H),
                 be1.reshape(1, H), W2)
  acc = _sc_agg(src, dst, zs)
  return _tc_head(acc, zs, dinv, c2.reshape(1, H), g2.reshape(1, H),
                  be2.reshape(1, H), b.reshape(N, 1).astype(jnp.int32),
                  Wh1, bh1.reshape(1, H), Wh2, bh2.reshape(1, DOUT))


# speedup vs baseline: 16.4438x; 1.2648x over previous
import functools

import jax
import jax.numpy as jnp
from jax import lax
from jax.experimental import pallas as pl
from jax.experimental.pallas import tpu as pltpu
from jax.experimental.pallas import tpu_sc as plsc

N = 10000
E = 160000
DIN = 256
H = 256
DOUT = 64
G = 64
EPS = 1e-5

NC = 2
NS = 16
L = 16
HH = H // 2
CHUNK = 128

BN = 1000
NB = N // BN

_mesh = lambda: plsc.VectorSubcoreMesh(core_axis_name="c", subcore_axis_name="s")


def _fill(ref, n, val):
  v = jnp.full((L,), val, jnp.float32)

  def body(i, _):
    ref[pl.ds(i * L, L)] = v
    return 0

  lax.fori_loop(0, n // L, body, 0)


def _sc_deg_body(dst_hbm, out_hbm, hist, ones_v, idx_v, zbuf):
  c = lax.axis_index("c")
  s = lax.axis_index("s")
  w = c * NS + s
  epw = 4992
  nfull = epw // CHUNK

  _fill(zbuf, CHUNK, 0.0)
  _fill(ones_v, CHUNK, 1.0)

  stripe = s * 624
  for k in range(5):
    pltpu.sync_copy(zbuf, hist.at[pl.ds(stripe + k * CHUNK, CHUNK)])
  plsc.subcore_barrier()

  base0 = w * epw

  def chunk(base):
    pltpu.sync_copy(dst_hbm.at[pl.ds(base, CHUNK)], idx_v)
    pltpu.sync_copy(ones_v, hist.at[idx_v], add=True)

  def body(i, _):
    chunk(base0 + i * CHUNK)
    return 0

  lax.fori_loop(0, nfull, body, 0)

  @pl.when(s == 0)
  def _():
    chunk(NC * NS * epw + c * CHUNK)

  plsc.subcore_barrier()

  @pl.when(s == 0)
  def _():
    pltpu.sync_copy(hist, out_hbm.at[c])


def _sc_deg(dst):
  return pl.kernel(
      _sc_deg_body,
      out_type=jax.ShapeDtypeStruct((NC, N), jnp.float32),
      mesh=_mesh(),
      scratch_types=[
          pltpu.VMEM_SHARED((N,), jnp.float32),
          pltpu.VMEM((CHUNK,), jnp.float32),
          pltpu.VMEM((CHUNK,), jnp.int32),
          pltpu.VMEM((CHUNK,), jnp.float32),
      ],
  )(dst)


def _sc_agg_body(src_hbm, dst_hbm, zs_hbm, out_hbm, acc, idx_s, idx_d,
                 rows, sem_i, sem_g, sem_sc):
  c = lax.axis_index("c")
  s = lax.axis_index("s")
  n = 78 + jnp.where(s < 2, 1, 0)
  r0 = 78 * s + jnp.minimum(s, 2)

  zv = jnp.zeros((L,), jnp.float32)

  def zbody(r, _):
    for jj in range(HH // L):
      rows[0, r, pl.ds(jj * L, L)] = zv
    return 0

  lax.fori_loop(0, CHUNK, zbody, 0)
  z0 = s * 624
  for k in range(5):
    pltpu.sync_copy(rows.at[0], acc.at[pl.ds(z0 + k * CHUNK, CHUNK)])
  plsc.subcore_barrier()

  def idx_stage(i):
    q = lax.rem(i, 4)
    pltpu.async_copy(src_hbm.at[r0 + i], idx_s.at[q], sem_i.at[q])
    pltpu.async_copy(dst_hbm.at[r0 + i], idx_d.at[q], sem_i.at[q])

  def idx_wait(i):
    q = lax.rem(i, 4)
    pltpu.make_async_copy(src_hbm.at[r0 + i], idx_s.at[q], sem_i.at[q]).wait()
    pltpu.make_async_copy(dst_hbm.at[r0 + i], idx_d.at[q], sem_i.at[q]).wait()

  def gather_fire(i, slot):
    q = lax.rem(i, 4)
    pltpu.async_copy(zs_hbm.at[c].at[idx_s.at[q]], rows.at[slot],
                     sem_g.at[slot])

  def gather_wait(i, slot):
    q = lax.rem(i, 4)
    pltpu.make_async_copy(zs_hbm.at[c].at[idx_s.at[q]], rows.at[slot],
                          sem_g.at[slot]).wait()

  def scatter_fire(i, slot):
    q = lax.rem(i, 4)
    pltpu.async_copy(rows.at[slot], acc.at[idx_d.at[q]], sem_sc.at[slot],
                     add=True)

  def scatter_wait(i, slot):
    q = lax.rem(i, 4)
    pltpu.make_async_copy(rows.at[slot], acc.at[idx_d.at[q]],
                          sem_sc.at[slot]).wait()

  idx_stage(0)
  idx_stage(1)
  idx_wait(0)
  gather_fire(0, 0)

  def body(i, _):
    slot = lax.rem(i, 2)
    nxt = lax.rem(i + 1, 2)

    @pl.when(i + 1 < n)
    def _():
      @pl.when(i >= 1)
      def _():
        scatter_wait(i - 1, nxt)
      idx_wait(i + 1)
      gather_fire(i + 1, nxt)

    gather_wait(i, slot)
    scatter_fire(i, slot)

    @pl.when(i + 2 < n)
    def _():
      idx_stage(i + 2)

    return 0

  lax.fori_loop(0, n, body, 0)

  scatter_wait(n - 2, lax.rem(n - 2, 2))
  scatter_wait(n - 1, lax.rem(n - 1, 2))

  plsc.subcore_barrier()
  ro = s * 624

  @pl.when(s < NS - 1)
  def _():
    pltpu.sync_copy(acc.at[pl.ds(ro, 624)], out_hbm.at[c].at[pl.ds(ro, 624)])

  @pl.when(s == NS - 1)
  def _():
    pltpu.sync_copy(acc.at[pl.ds(9360, 640)],
                    out_hbm.at[c].at[pl.ds(9360, 640)])


def _sc_agg(src2, dst2, zs):
  return pl.kernel(
      _sc_agg_body,
      out_type=jax.ShapeDtypeStruct((NC, N, HH), jnp.float32),
      mesh=_mesh(),
      scratch_types=[
          pltpu.VMEM_SHARED((N, HH), jnp.float32),
          pltpu.VMEM((4, CHUNK), jnp.int32),
          pltpu.VMEM((4, CHUNK), jnp.int32),
          pltpu.VMEM((2, CHUNK, HH), jnp.float32),
          pltpu.SemaphoreType.DMA((4,)),
          pltpu.SemaphoreType.DMA((2,)),
          pltpu.SemaphoreType.DMA((2,)),
      ],
  )(src2, dst2, zs)


def _tc_prep_body(x_ref, w_ref, da_ref, db_ref, zs_ref, dinv_ref):
  dinv = lax.rsqrt(da_ref[...] + db_ref[...] + 1.0)
  z = lax.dot_general(x_ref[...], w_ref[...], (((1,), (1,)), ((), ())),
                      preferred_element_type=jnp.float32)
  zs = z * dinv
  zs_ref[0] = zs[:, :HH]
  zs_ref[1] = zs[:, HH:]
  dinv_ref[...] = dinv


def _tc_prep(x, W0, degA, degB):
  return pl.pallas_call(
      _tc_prep_body,
      grid=(NB,),
      in_specs=[
          pl.BlockSpec((BN, DIN), lambda j: (j, 0)),
          pl.BlockSpec((H, DIN), lambda j: (0, 0)),
          pl.BlockSpec((BN, 1), lambda j: (j, 0)),
          pl.BlockSpec((BN, 1), lambda j: (j, 0)),
      ],
      out_specs=[
          pl.BlockSpec((NC, BN, HH), lambda j: (0, j, 0)),
          pl.BlockSpec((BN, 1), lambda j: (j, 0)),
      ],
      out_shape=[
          jax.ShapeDtypeStruct((NC, N, HH), jnp.float32),
          jax.ShapeDtypeStruct((N, 1), jnp.float32),
      ],
  )(x, W0, degA, degB)


def _pre_act(acc_ref, zs_ref, dinv_ref, c_ref):
  a = jnp.concatenate([acc_ref[0] + zs_ref[0], acc_ref[1] + zs_ref[1]],
                      axis=-1)
  return a * dinv_ref[...] + c_ref[...]


def _bn_relu(t, sums_ref, sumsq_ref, g_ref, be_ref):
  mean = sums_ref[...] * (1.0 / N)
  var = sumsq_ref[...] * (1.0 / N) - mean * mean
  return jnp.maximum((t - mean) * lax.rsqrt(var + EPS) * g_ref[...]
                     + be_ref[...], 0.0)


def _tc_layer_body(acc_ref, zs_ref, dinv_ref, c_ref, g_ref, be_ref, w_ref,
                   out_ref, sums_ref, sumsq_ref):
  p = pl.program_id(0)
  j = pl.program_id(1)
  t = _pre_act(acc_ref, zs_ref, dinv_ref, c_ref)

  @pl.when(p == 0)
  def _():
    @pl.when(j == 0)
    def _():
      sums_ref[...] = jnp.zeros_like(sums_ref)
      sumsq_ref[...] = jnp.zeros_like(sumsq_ref)

    sums_ref[...] += jnp.sum(t, axis=0, keepdims=True)
    sumsq_ref[...] += jnp.sum(t * t, axis=0, keepdims=True)

  @pl.when(p == 1)
  def _():
    h = _bn_relu(t, sums_ref, sumsq_ref, g_ref, be_ref)
    z = lax.dot_general(h, w_ref[...], (((1,), (1,)), ((), ())),
                        preferred_element_type=jnp.float32)
    zsn = z * dinv_ref[...]
    out_ref[0] = zsn[:, :HH]
    out_ref[1] = zsn[:, HH:]


def _tc_layer(acc, zs, dinv, c, g, be, Wn):
  return pl.pallas_call(
      _tc_layer_body,
      grid=(2, NB),
      in_specs=[
          pl.BlockSpec((NC, BN, HH), lambda p, j: (0, j, 0)),
          pl.BlockSpec((NC, BN, HH), lambda p, j: (0, j, 0)),
          pl.BlockSpec((BN, 1), lambda p, j: (j, 0)),
          pl.BlockSpec((1, H), lambda p, j: (0, 0)),
          pl.BlockSpec((1, H), lambda p, j: (0, 0)),
          pl.BlockSpec((1, H), lambda p, j: (0, 0)),
          pl.BlockSpec((H, H), lambda p, j: (0, 0)),
      ],
      out_specs=pl.BlockSpec((NC, BN, HH), lambda p, j: (0, j, 0)),
      out_shape=jax.ShapeDtypeStruct((NC, N, HH), jnp.float32),
      scratch_shapes=[
          pltpu.VMEM((1, H), jnp.float32),
          pltpu.VMEM((1, H), jnp.float32),
      ],
  )(acc, zs, dinv, c, g, be, Wn)


def _tc_head_body(acc_ref, zs_ref, dinv_ref, c_ref, g_ref, be_ref, b_ref,
                  wh1_ref, bh1_ref, wh2_ref, bh2_ref, out_ref,
                  sums_ref, sumsq_ref, psum_ref, pcnt_ref):
  p = pl.program_id(0)
  j = pl.program_id(1)

  @pl.when(p < 2)
  def _():
    t = _pre_act(acc_ref, zs_ref, dinv_ref, c_ref)

    @pl.when(p == 0)
    def _():
      @pl.when(j == 0)
      def _():
        sums_ref[...] = jnp.zeros_like(sums_ref)
        sumsq_ref[...] = jnp.zeros_like(sumsq_ref)

      sums_ref[...] += jnp.sum(t, axis=0, keepdims=True)
      sumsq_ref[...] += jnp.sum(t * t, axis=0, keepdims=True)

    @pl.when(p == 1)
    def _():
      @pl.when(j == 0)
      def _():
        psum_ref[...] = jnp.zeros_like(psum_ref)
        pcnt_ref[...] = jnp.zeros_like(pcnt_ref)

      h = _bn_relu(t, sums_ref, sumsq_ref, g_ref, be_ref)
      gids = lax.broadcasted_iota(jnp.int32, (1, G), 1)
      mask = (b_ref[...] == gids).astype(jnp.float32)
      psum_ref[...] += lax.dot_general(
          mask, h, (((0,), (0,)), ((), ())),
          preferred_element_type=jnp.float32)
      pcnt_ref[...] += lax.dot_general(
          mask, jnp.ones((BN, 1), jnp.float32), (((0,), (0,)), ((), ())),
          preferred_element_type=jnp.float32)

  @pl.when((p == 2) & (j == 0))
  def _():
    pool = psum_ref[...] / jnp.maximum(pcnt_ref[...], 1.0)
    hid = jnp.maximum(
        lax.dot_general(pool, wh1_ref[...], (((1,), (1,)), ((), ())),
                        preferred_element_type=jnp.float32) + bh1_ref[...],
        0.0)
    out_ref[...] = lax.dot_general(
        hid, wh2_ref[...], (((1,), (1,)), ((), ())),
        preferred_element_type=jnp.float32) + bh2_ref[...]


def _tc_head(acc, zs, dinv, c, g, be, b2, Wh1, bh1, Wh2, bh2):
  return pl.pallas_call(
      _tc_head_body,
      grid=(3, NB),
      in_specs=[
          pl.BlockSpec((NC, BN, HH), lambda p, j: (0, j, 0)),
          pl.BlockSpec((NC, BN, HH), lambda p, j: (0, j, 0)),
          pl.BlockSpec((BN, 1), lambda p, j: (j, 0)),
          pl.BlockSpec((1, H), lambda p, j: (0, 0)),
          pl.BlockSpec((1, H), lambda p, j: (0, 0)),
          pl.BlockSpec((1, H), lambda p, j: (0, 0)),
          pl.BlockSpec((BN, 1), lambda p, j: (j, 0)),
          pl.BlockSpec((H, H), lambda p, j: (0, 0)),
          pl.BlockSpec((1, H), lambda p, j: (0, 0)),
          pl.BlockSpec((DOUT, H), lambda p, j: (0, 0)),
          pl.BlockSpec((1, DOUT), lambda p, j: (0, 0)),
      ],
      out_specs=pl.BlockSpec((G, DOUT), lambda p, j: (0, 0)),
      out_shape=jax.ShapeDtypeStruct((G, DOUT), jnp.float32),
      scratch_shapes=[
          pltpu.VMEM((1, H), jnp.float32),
          pltpu.VMEM((1, H), jnp.float32),
          pltpu.VMEM((G, H), jnp.float32),
          pltpu.VMEM((G, 1), jnp.float32),
      ],
  )(acc, zs, dinv, c, g, be, b2, Wh1, bh1, Wh2, bh2)


def kernel(x, ei, b, W0, c0, W1, c1, W2, c2, g0, be0, g1, be1, g2, be2,
           Wh1, bh1, Wh2, bh2):
  src = ei[0]
  dst = ei[1]
  src2 = src.reshape(E // CHUNK, CHUNK)
  dst2 = dst.reshape(E // CHUNK, CHUNK)
  deg2 = _sc_deg(dst)
  degA = deg2[0].reshape(N, 1)
  degB = deg2[1].reshape(N, 1)
  zs, dinv = _tc_prep(x, W0, degA, degB)
  acc = _sc_agg(src2, dst2, zs)
  zs = _tc_layer(acc, zs, dinv, c0.reshape(1, H), g0.reshape(1, H),
                 be0.reshape(1, H), W1)
  acc = _sc_agg(src2, dst2, zs)
  zs = _tc_layer(acc, zs, dinv, c1.reshape(1, H), g1.reshape(1, H),
                 be1.reshape(1, H), W2)
  acc = _sc_agg(src2, dst2, zs)
  return _tc_head(acc, zs, dinv, c2.reshape(1, H), g2.reshape(1, H),
                  be2.reshape(1, H), b.reshape(N, 1).astype(jnp.int32),
                  Wh1, bh1.reshape(1, H), Wh2, bh2.reshape(1, DOUT))


# speedup vs baseline: 17.0477x; 1.0367x over previous
import functools

import jax
import jax.numpy as jnp
from jax import lax
from jax.experimental import pallas as pl
from jax.experimental.pallas import tpu as pltpu
from jax.experimental.pallas import tpu_sc as plsc

N = 10000
E = 160000
DIN = 256
H = 256
DOUT = 64
G = 64
EPS = 1e-5

NC = 2
NS = 16
L = 16
HH = H // 2
CHUNK = 128

BN = 1000
NB = N // BN

_mesh = lambda: plsc.VectorSubcoreMesh(core_axis_name="c", subcore_axis_name="s")


def _fill(ref, n, val):
  v = jnp.full((L,), val, jnp.float32)

  def body(i, _):
    ref[pl.ds(i * L, L)] = v
    return 0

  lax.fori_loop(0, n // L, body, 0)


def _sc_deg_body(dst_hbm, out_hbm, hist, ones_v, idx_v, zbuf):
  c = lax.axis_index("c")
  s = lax.axis_index("s")
  w = c * NS + s
  epw = 4992
  nfull = epw // CHUNK

  _fill(zbuf, CHUNK, 0.0)
  _fill(ones_v, CHUNK, 1.0)

  stripe = s * 624
  for k in range(5):
    pltpu.sync_copy(zbuf, hist.at[pl.ds(stripe + k * CHUNK, CHUNK)])
  plsc.subcore_barrier()

  base0 = w * epw

  def chunk(base):
    pltpu.sync_copy(dst_hbm.at[pl.ds(base, CHUNK)], idx_v)
    pltpu.sync_copy(ones_v, hist.at[idx_v], add=True)

  def body(i, _):
    chunk(base0 + i * CHUNK)
    return 0

  lax.fori_loop(0, nfull, body, 0)

  @pl.when(s == 0)
  def _():
    chunk(NC * NS * epw + c * CHUNK)

  plsc.subcore_barrier()

  @pl.when(s == 0)
  def _():
    pltpu.sync_copy(hist, out_hbm.at[c])


def _sc_deg(dst):
  return pl.kernel(
      _sc_deg_body,
      out_type=jax.ShapeDtypeStruct((NC, N), jnp.float32),
      mesh=_mesh(),
      scratch_types=[
          pltpu.VMEM_SHARED((N,), jnp.float32),
          pltpu.VMEM((CHUNK,), jnp.float32),
          pltpu.VMEM((CHUNK,), jnp.int32),
          pltpu.VMEM((CHUNK,), jnp.float32),
      ],
  )(dst)


def _sc_agg_body(src_hbm, dst_hbm, zs_hbm, out_hbm, acc, idx_s, idx_d,
                 rows, sem_i, sem_g, sem_sc):
  c = lax.axis_index("c")
  s = lax.axis_index("s")
  n = 78 + jnp.where(s < 2, 1, 0)
  r0 = 78 * s + jnp.minimum(s, 2)

  zv = jnp.zeros((L,), jnp.float32)

  def zbody(r, _):
    for jj in range(HH // L):
      rows[0, r, pl.ds(jj * L, L)] = zv
    return 0

  lax.fori_loop(0, CHUNK, zbody, 0)
  z0 = s * 624
  for k in range(5):
    pltpu.sync_copy(rows.at[0], acc.at[pl.ds(z0 + k * CHUNK, CHUNK)])
  plsc.subcore_barrier()

  def idx_stage(i):
    q = lax.rem(i, 4)
    pltpu.async_copy(src_hbm.at[r0 + i], idx_s.at[q], sem_i.at[q])
    pltpu.async_copy(dst_hbm.at[r0 + i], idx_d.at[q], sem_i.at[q])

  def idx_wait(i):
    q = lax.rem(i, 4)
    pltpu.make_async_copy(src_hbm.at[r0 + i], idx_s.at[q], sem_i.at[q]).wait()
    pltpu.make_async_copy(dst_hbm.at[r0 + i], idx_d.at[q], sem_i.at[q]).wait()

  def gather_fire(i, slot):
    q = lax.rem(i, 4)
    pltpu.async_copy(zs_hbm.at[c].at[idx_s.at[q]], rows.at[slot],
                     sem_g.at[slot])

  def gather_wait(i, slot):
    q = lax.rem(i, 4)
    pltpu.make_async_copy(zs_hbm.at[c].at[idx_s.at[q]], rows.at[slot],
                          sem_g.at[slot]).wait()

  def scatter_fire(i, slot):
    q = lax.rem(i, 4)
    pltpu.async_copy(rows.at[slot], acc.at[idx_d.at[q]], sem_sc.at[slot],
                     add=True)

  def scatter_wait(i, slot):
    q = lax.rem(i, 4)
    pltpu.make_async_copy(rows.at[slot], acc.at[idx_d.at[q]],
                          sem_sc.at[slot]).wait()

  idx_stage(0)
  idx_stage(1)
  idx_wait(0)
  gather_fire(0, 0)

  def body(i, _):
    slot = lax.rem(i, 2)
    nxt = lax.rem(i + 1, 2)

    @pl.when(i + 1 < n)
    def _():
      @pl.when(i >= 1)
      def _():
        scatter_wait(i - 1, nxt)
      idx_wait(i + 1)
      gather_fire(i + 1, nxt)

    gather_wait(i, slot)
    scatter_fire(i, slot)

    @pl.when(i + 2 < n)
    def _():
      idx_stage(i + 2)

    return 0

  lax.fori_loop(0, n, body, 0)

  scatter_wait(n - 2, lax.rem(n - 2, 2))
  scatter_wait(n - 1, lax.rem(n - 1, 2))

  plsc.subcore_barrier()
  ro = s * 624

  @pl.when(s < NS - 1)
  def _():
    pltpu.sync_copy(acc.at[pl.ds(ro, 624)], out_hbm.at[c].at[pl.ds(ro, 624)])

  @pl.when(s == NS - 1)
  def _():
    pltpu.sync_copy(acc.at[pl.ds(9360, 640)],
                    out_hbm.at[c].at[pl.ds(9360, 640)])


def _sc_agg(src2, dst2, zs):
  return pl.kernel(
      _sc_agg_body,
      out_type=jax.ShapeDtypeStruct((NC, N, HH), jnp.float32),
      mesh=_mesh(),
      scratch_types=[
          pltpu.VMEM_SHARED((N, HH), jnp.float32),
          pltpu.VMEM((4, CHUNK), jnp.int32),
          pltpu.VMEM((4, CHUNK), jnp.int32),
          pltpu.VMEM((2, CHUNK, HH), jnp.float32),
          pltpu.SemaphoreType.DMA((4,)),
          pltpu.SemaphoreType.DMA((2,)),
          pltpu.SemaphoreType.DMA((2,)),
      ],
  )(src2, dst2, zs)


def _tc_prep_body(x_ref, w_ref, da_ref, db_ref, zs_ref, dinv_ref):
  dinv = lax.rsqrt(da_ref[...] + db_ref[...] + 1.0)
  z = lax.dot_general(x_ref[...], w_ref[...], (((1,), (1,)), ((), ())),
                      preferred_element_type=jnp.float32)
  zs = z * dinv
  zs_ref[0] = zs[:, :HH]
  zs_ref[1] = zs[:, HH:]
  dinv_ref[...] = dinv


def _tc_prep(x, W0, degA, degB):
  return pl.pallas_call(
      _tc_prep_body,
      grid=(NB,),
      in_specs=[
          pl.BlockSpec((BN, DIN), lambda j: (j, 0)),
          pl.BlockSpec((H, DIN), lambda j: (0, 0)),
          pl.BlockSpec((BN, 1), lambda j: (j, 0)),
          pl.BlockSpec((BN, 1), lambda j: (j, 0)),
      ],
      out_specs=[
          pl.BlockSpec((NC, BN, HH), lambda j: (0, j, 0)),
          pl.BlockSpec((BN, 1), lambda j: (j, 0)),
      ],
      out_shape=[
          jax.ShapeDtypeStruct((NC, N, HH), jnp.float32),
          jax.ShapeDtypeStruct((N, 1), jnp.float32),
      ],
  )(x, W0, degA, degB)


def _pre_act(acc_ref, zs_ref, dinv_ref, c_ref):
  a = jnp.concatenate([acc_ref[0] + zs_ref[0], acc_ref[1] + zs_ref[1]],
                      axis=-1)
  return a * dinv_ref[...] + c_ref[...]


def _bn_relu(t, sums_ref, sumsq_ref, g_ref, be_ref):
  mean = sums_ref[...] * (1.0 / N)
  var = sumsq_ref[...] * (1.0 / N) - mean * mean
  return jnp.maximum((t - mean) * lax.rsqrt(var + EPS) * g_ref[...]
                     + be_ref[...], 0.0)


def _tc_layer_body(acc_ref, zs_ref, dinv_ref, c_ref, g_ref, be_ref, w_ref,
                   out_ref, sums_ref, sumsq_ref, t_ref):
  p = pl.program_id(0)
  j = pl.program_id(1)

  @pl.when(p == 0)
  def _():
    t = _pre_act(acc_ref, zs_ref, dinv_ref, c_ref)
    t_ref[j] = t

    @pl.when(j == 0)
    def _():
      sums_ref[...] = jnp.zeros_like(sums_ref)
      sumsq_ref[...] = jnp.zeros_like(sumsq_ref)

    sums_ref[...] += jnp.sum(t, axis=0, keepdims=True)
    sumsq_ref[...] += jnp.sum(t * t, axis=0, keepdims=True)

  @pl.when(p == 1)
  def _():
    h = _bn_relu(t_ref[j], sums_ref, sumsq_ref, g_ref, be_ref)
    z = lax.dot_general(h, w_ref[...], (((1,), (1,)), ((), ())),
                        preferred_element_type=jnp.float32)
    zsn = z * dinv_ref[...]
    out_ref[0] = zsn[:, :HH]
    out_ref[1] = zsn[:, HH:]


def _tc_layer(acc, zs, dinv, c, g, be, Wn):
  return pl.pallas_call(
      _tc_layer_body,
      grid=(2, NB),
      in_specs=[
          pl.BlockSpec((NC, BN, HH),
                       lambda p, j: (0, jnp.where(p == 0, j, 0), 0)),
          pl.BlockSpec((NC, BN, HH),
                       lambda p, j: (0, jnp.where(p == 0, j, 0), 0)),
          pl.BlockSpec((BN, 1), lambda p, j: (j, 0)),
          pl.BlockSpec((1, H), lambda p, j: (0, 0)),
          pl.BlockSpec((1, H), lambda p, j: (0, 0)),
          pl.BlockSpec((1, H), lambda p, j: (0, 0)),
          pl.BlockSpec((H, H), lambda p, j: (0, 0)),
      ],
      out_specs=pl.BlockSpec((NC, BN, HH), lambda p, j: (0, j, 0)),
      out_shape=jax.ShapeDtypeStruct((NC, N, HH), jnp.float32),
      scratch_shapes=[
          pltpu.VMEM((1, H), jnp.float32),
          pltpu.VMEM((1, H), jnp.float32),
          pltpu.VMEM((NB, BN, H), jnp.float32),
      ],
  )(acc, zs, dinv, c, g, be, Wn)


def _tc_head_body(acc_ref, zs_ref, dinv_ref, c_ref, g_ref, be_ref, b_ref,
                  wh1_ref, bh1_ref, wh2_ref, bh2_ref, out_ref,
                  sums_ref, sumsq_ref, psum_ref, pcnt_ref, t_ref):
  p = pl.program_id(0)
  j = pl.program_id(1)

  @pl.when(p < 2)
  def _():
    @pl.when(p == 0)
    def _():
      t = _pre_act(acc_ref, zs_ref, dinv_ref, c_ref)
      t_ref[j] = t

      @pl.when(j == 0)
      def _():
        sums_ref[...] = jnp.zeros_like(sums_ref)
        sumsq_ref[...] = jnp.zeros_like(sumsq_ref)

      sums_ref[...] += jnp.sum(t, axis=0, keepdims=True)
      sumsq_ref[...] += jnp.sum(t * t, axis=0, keepdims=True)

    @pl.when(p == 1)
    def _():
      @pl.when(j == 0)
      def _():
        psum_ref[...] = jnp.zeros_like(psum_ref)
        pcnt_ref[...] = jnp.zeros_like(pcnt_ref)

      h = _bn_relu(t_ref[j], sums_ref, sumsq_ref, g_ref, be_ref)
      gids = lax.broadcasted_iota(jnp.int32, (1, G), 1)
      mask = (b_ref[...] == gids).astype(jnp.float32)
      psum_ref[...] += lax.dot_general(
          mask, h, (((0,), (0,)), ((), ())),
          preferred_element_type=jnp.float32)
      pcnt_ref[...] += lax.dot_general(
          mask, jnp.ones((BN, 1), jnp.float32), (((0,), (0,)), ((), ())),
          preferred_element_type=jnp.float32)

  @pl.when((p == 2) & (j == 0))
  def _():
    pool = psum_ref[...] / jnp.maximum(pcnt_ref[...], 1.0)
    hid = jnp.maximum(
        lax.dot_general(pool, wh1_ref[...], (((1,), (1,)), ((), ())),
                        preferred_element_type=jnp.float32) + bh1_ref[...],
        0.0)
    out_ref[...] = lax.dot_general(
        hid, wh2_ref[...], (((1,), (1,)), ((), ())),
        preferred_element_type=jnp.float32) + bh2_ref[...]


def _tc_head(acc, zs, dinv, c, g, be, b2, Wh1, bh1, Wh2, bh2):
  return pl.pallas_call(
      _tc_head_body,
      grid=(3, NB),
      in_specs=[
          pl.BlockSpec((NC, BN, HH),
                       lambda p, j: (0, jnp.where(p == 0, j, 0), 0)),
          pl.BlockSpec((NC, BN, HH),
                       lambda p, j: (0, jnp.where(p == 0, j, 0), 0)),
          pl.BlockSpec((BN, 1), lambda p, j: (j, 0)),
          pl.BlockSpec((1, H), lambda p, j: (0, 0)),
          pl.BlockSpec((1, H), lambda p, j: (0, 0)),
          pl.BlockSpec((1, H), lambda p, j: (0, 0)),
          pl.BlockSpec((BN, 1), lambda p, j: (j, 0)),
          pl.BlockSpec((H, H), lambda p, j: (0, 0)),
          pl.BlockSpec((1, H), lambda p, j: (0, 0)),
          pl.BlockSpec((DOUT, H), lambda p, j: (0, 0)),
          pl.BlockSpec((1, DOUT), lambda p, j: (0, 0)),
      ],
      out_specs=pl.BlockSpec((G, DOUT), lambda p, j: (0, 0)),
      out_shape=jax.ShapeDtypeStruct((G, DOUT), jnp.float32),
      scratch_shapes=[
          pltpu.VMEM((1, H), jnp.float32),
          pltpu.VMEM((1, H), jnp.float32),
          pltpu.VMEM((G, H), jnp.float32),
          pltpu.VMEM((G, 1), jnp.float32),
          pltpu.VMEM((NB, BN, H), jnp.float32),
      ],
  )(acc, zs, dinv, c, g, be, b2, Wh1, bh1, Wh2, bh2)


def kernel(x, ei, b, W0, c0, W1, c1, W2, c2, g0, be0, g1, be1, g2, be2,
           Wh1, bh1, Wh2, bh2):
  src = ei[0]
  dst = ei[1]
  src2 = src.reshape(E // CHUNK, CHUNK)
  dst2 = dst.reshape(E // CHUNK, CHUNK)
  deg2 = _sc_deg(dst)
  degA = deg2[0].reshape(N, 1)
  degB = deg2[1].reshape(N, 1)
  zs, dinv = _tc_prep(x, W0, degA, degB)
  acc = _sc_agg(src2, dst2, zs)
  zs = _tc_layer(acc, zs, dinv, c0.reshape(1, H), g0.reshape(1, H),
                 be0.reshape(1, H), W1)
  acc = _sc_agg(src2, dst2, zs)
  zs = _tc_layer(acc, zs, dinv, c1.reshape(1, H), g1.reshape(1, H),
                 be1.reshape(1, H), W2)
  acc = _sc_agg(src2, dst2, zs)
  return _tc_head(acc, zs, dinv, c2.reshape(1, H), g2.reshape(1, H),
                  be2.reshape(1, H), b.reshape(N, 1).astype(jnp.int32),
                  Wh1, bh1.reshape(1, H), Wh2, bh2.reshape(1, DOUT))
